# Initial kernel scaffold; baseline (speedup 1.0000x reference)
import jax
import jax.numpy as jnp
from jax.experimental import pallas as pl


def kernel(x, table):
    raise NotImplementedError("write your pallas kernel here")


# speedup vs baseline: 2.4221x; 2.4221x over previous
import functools
import math

import jax
import jax.numpy as jnp
from jax import lax
from jax.experimental import pallas as pl
from jax.experimental.pallas import tpu as pltpu
from jax.experimental.pallas import tpu_sc as plsc

_VOCAB = 100000
_DIM = 128
_B = 4096
_L = 50
_TOT = _B * _L
_NC = 2
_NS = 16
_NW = _NC * _NS
_BPW = _TOT // _NW
_C = 128
_G = _BPW // _C
_LANES = 16
_SCALE = math.sqrt(_DIM)


def _build_sc_kernel():
    mesh = plsc.VectorSubcoreMesh(core_axis_name="c", subcore_axis_name="s")

    @functools.partial(
        pl.kernel,
        mesh=mesh,
        out_type=jax.ShapeDtypeStruct((_TOT, _DIM), jnp.float32),
        scratch_types=[
            pltpu.VMEM((_G, _C), jnp.int32),
            pltpu.VMEM((_C, _DIM), jnp.float32),
            pltpu.SemaphoreType.DMA,
        ],
    )
    def k(table_hbm, x_hbm, out_hbm, idx_v, rows_v, sem):
        wid = lax.axis_index("s") * _NC + lax.axis_index("c")
        base = wid * _BPW
        pltpu.sync_copy(x_hbm.at[wid], idx_v)

        def chunk_body(g, _):
            pltpu.async_copy(table_hbm.at[idx_v.at[g]], rows_v, sem).wait()

            def row_body(r, _):
                for c in range(_DIM // _LANES):
                    sl = pl.ds(c * _LANES, _LANES)
                    rows_v[r, sl] = rows_v[r, sl] * _SCALE
                return 0

            lax.fori_loop(0, _C, row_body, 0)
            pltpu.sync_copy(rows_v, out_hbm.at[pl.ds(base + g * _C, _C)])
            return 0

        lax.fori_loop(0, _G, chunk_body, 0)

    return k


_sc_gather = _build_sc_kernel()


def kernel(x, table):
    x3d = x.reshape(_NW, _G, _C)
    out = _sc_gather(table, x3d)
    return out.reshape(_B, _L, _DIM)


# speedup vs baseline: 2.8944x; 1.1950x over previous
import functools
import math

import jax
import jax.numpy as jnp
from jax import lax
from jax.experimental import pallas as pl
from jax.experimental.pallas import tpu as pltpu
from jax.experimental.pallas import tpu_sc as plsc

_VOCAB = 100000
_DIM = 128
_B = 4096
_L = 50
_TOT = _B * _L
_NC = 2
_NS = 16
_NW = _NC * _NS
_BPW = _TOT // _NW
_C = 128
_G = _BPW // _C
_LANES = 16
_SCALE = math.sqrt(_DIM)


def _build_sc_kernel():
    mesh = plsc.VectorSubcoreMesh(core_axis_name="c", subcore_axis_name="s")

    @functools.partial(
        pl.kernel,
        mesh=mesh,
        out_type=jax.ShapeDtypeStruct((_TOT, _DIM), jnp.float32),
        scratch_types=[
            pltpu.VMEM((_G, _C), jnp.int32),
            pltpu.VMEM((2, _C, _DIM), jnp.float32),
            pltpu.SemaphoreType.DMA,
            pltpu.SemaphoreType.DMA,
        ],
    )
    def k(table_hbm, x_hbm, out_hbm, idx_v, rows_v, gsem, osem):
        wid = lax.axis_index("s") * _NC + lax.axis_index("c")
        base = wid * _BPW
        pltpu.sync_copy(x_hbm.at[wid], idx_v)
        pltpu.async_copy(table_hbm.at[idx_v.at[0]], rows_v.at[0], gsem)

        def outer(h, _):
            for b in range(2):
                g = 2 * h + b
                nb = 1 - b

                @pl.when(g > 0)
                def _wait_prev_write():
                    pltpu.make_async_copy(
                        rows_v.at[nb], out_hbm.at[pl.ds(base, _C)], osem
                    ).wait()

                @pl.when(g + 1 < _G)
                def _start_next_gather():
                    pltpu.async_copy(
                        table_hbm.at[idx_v.at[g + 1]], rows_v.at[nb], gsem
                    )

                pltpu.make_async_copy(
                    table_hbm.at[idx_v.at[g]], rows_v.at[b], gsem
                ).wait()

                def row_body(r, _):
                    for c in range(_DIM // _LANES):
                        sl = pl.ds(c * _LANES, _LANES)
                        rows_v[b, r, sl] = rows_v[b, r, sl] * _SCALE
                    return 0

                lax.fori_loop(0, _C, row_body, 0)
                pltpu.async_copy(
                    rows_v.at[b], out_hbm.at[pl.ds(base + g * _C, _C)], osem
                )
            return 0

        lax.fori_loop(0, _G // 2, outer, 0)
        pltpu.make_async_copy(
            rows_v.at[1], out_hbm.at[pl.ds(base, _C)], osem
        ).wait()

    return k


_sc_gather = _build_sc_kernel()


def kernel(x, table):
    x3d = x.reshape(_NW, _G, _C)
    out = _sc_gather(table, x3d)
    return out.reshape(_B, _L, _DIM)


# speedup vs baseline: 2.8977x; 1.0011x over previous
import functools
import math

import jax
import jax.numpy as jnp
from jax import lax
from jax.experimental import pallas as pl
from jax.experimental.pallas import tpu as pltpu
from jax.experimental.pallas import tpu_sc as plsc

_VOCAB = 100000
_DIM = 128
_B = 4096
_L = 50
_TOT = _B * _L
_NC = 2
_NS = 16
_NW = _NC * _NS
_BPW = _TOT // _NW
_C = 128
_G = _BPW // _C
_LANES = 16
_SCALE = math.sqrt(_DIM)


def _build_sc_kernel():
    mesh = plsc.VectorSubcoreMesh(core_axis_name="c", subcore_axis_name="s")

    @functools.partial(
        pl.kernel,
        mesh=mesh,
        out_type=jax.ShapeDtypeStruct((_TOT, _DIM), jnp.float32),
        scratch_types=[
            pltpu.VMEM((_G, _C), jnp.int32),
            pltpu.VMEM((2, _C, _DIM), jnp.float32),
            pltpu.SemaphoreType.DMA,
            pltpu.SemaphoreType.DMA,
        ],
    )
    def k(table_hbm, x_hbm, out_hbm, idx_v, rows_v, gsem, osem):
        wid = lax.axis_index("s") * _NC + lax.axis_index("c")
        base = wid * _BPW
        pltpu.sync_copy(x_hbm.at[wid], idx_v)
        pltpu.async_copy(table_hbm.at[idx_v.at[0]], rows_v.at[0], gsem)

        def outer(h, _):
            for b in range(2):
                g = 2 * h + b
                nb = 1 - b

                @pl.when(g > 0)
                def _wait_prev_write():
                    pltpu.make_async_copy(
                        rows_v.at[nb], out_hbm.at[pl.ds(base, _C)], osem
                    ).wait()

                @pl.when(g + 1 < _G)
                def _start_next_gather():
                    pltpu.async_copy(
                        table_hbm.at[idx_v.at[g + 1]], rows_v.at[nb], gsem
                    )

                pltpu.make_async_copy(
                    table_hbm.at[idx_v.at[g]], rows_v.at[b], gsem
                ).wait()

                @plsc.parallel_loop(0, _C * _DIM // _LANES, unroll=8)
                def _scale(j):
                    r = lax.shift_right_logical(j, 3)
                    sl = pl.ds(lax.shift_left(j & 7, 4), _LANES)
                    rows_v[b, r, sl] = rows_v[b, r, sl] * _SCALE
                pltpu.async_copy(
                    rows_v.at[b], out_hbm.at[pl.ds(base + g * _C, _C)], osem
                )
            return 0

        lax.fori_loop(0, _G // 2, outer, 0)
        pltpu.make_async_copy(
            rows_v.at[1], out_hbm.at[pl.ds(base, _C)], osem
        ).wait()

    return k


_sc_gather = _build_sc_kernel()


def kernel(x, table):
    x3d = x.reshape(_NW, _G, _C)
    out = _sc_gather(table, x3d)
    return out.reshape(_B, _L, _DIM)


# speedup vs baseline: 4.3044x; 1.4855x over previous
import functools
import math

import jax
import jax.numpy as jnp
from jax import lax
from jax.experimental import pallas as pl
from jax.experimental.pallas import tpu as pltpu
from jax.experimental.pallas import tpu_sc as plsc

_VOCAB = 100000
_DIM = 128
_B = 4096
_L = 50
_NC = 2
_NS = 16
_NW = _NC * _NS
_RPW = _B // _NW
_LANES = 16
_SCALE = math.sqrt(_DIM)


def _build_sc_kernel():
    mesh = plsc.VectorSubcoreMesh(core_axis_name="c", subcore_axis_name="s")

    @functools.partial(
        pl.kernel,
        mesh=mesh,
        out_type=jax.ShapeDtypeStruct((_B, _L, _DIM), jnp.float32),
        scratch_types=[
            pltpu.VMEM((_RPW, _L), jnp.int32),
            pltpu.VMEM((2, _L, _DIM), jnp.float32),
            pltpu.SemaphoreType.DMA,
            pltpu.SemaphoreType.DMA,
        ],
    )
    def k(table_hbm, x_hbm, out_hbm, idx_v, rows_v, gsem, osem):
        wid = lax.axis_index("s") * _NC + lax.axis_index("c")
        base = wid * _RPW
        pltpu.sync_copy(x_hbm.at[pl.ds(base, _RPW)], idx_v)
        pltpu.async_copy(table_hbm.at[idx_v.at[0]], rows_v.at[0], gsem)

        def outer(h, _):
            for b in range(2):
                g = 2 * h + b
                nb = 1 - b

                @pl.when(g > 0)
                def _wait_prev_write():
                    pltpu.make_async_copy(
                        rows_v.at[nb], out_hbm.at[base], osem
                    ).wait()

                @pl.when(g + 1 < _RPW)
                def _start_next_gather():
                    pltpu.async_copy(
                        table_hbm.at[idx_v.at[g + 1]], rows_v.at[nb], gsem
                    )

                pltpu.make_async_copy(
                    table_hbm.at[idx_v.at[g]], rows_v.at[b], gsem
                ).wait()

                @plsc.parallel_loop(0, _L * _DIM // _LANES, unroll=8)
                def _scale(j):
                    r = lax.shift_right_logical(j, 3)
                    sl = pl.ds(lax.shift_left(j & 7, 4), _LANES)
                    rows_v[b, r, sl] = rows_v[b, r, sl] * _SCALE

                pltpu.async_copy(
                    rows_v.at[b], out_hbm.at[base + g], osem
                )
            return 0

        lax.fori_loop(0, _RPW // 2, outer, 0)
        pltpu.make_async_copy(rows_v.at[1], out_hbm.at[base], osem).wait()

    return k


_sc_gather = _build_sc_kernel()


def kernel(x, table):
    return _sc_gather(table, x)


# speedup vs baseline: 5.1563x; 1.1979x over previous
import functools
import math

import jax
import jax.numpy as jnp
from jax import lax
from jax.experimental import pallas as pl
from jax.experimental.pallas import tpu as pltpu
from jax.experimental.pallas import tpu_sc as plsc

_VOCAB = 100000
_DIM = 128
_B = 4096
_L = 50
_NC = 2
_NS = 16
_NW = _NC * _NS
_RPW = _B // _NW
_LANES = 16
_SCALE = math.sqrt(_DIM)


def _build_sc_kernel():
    mesh = plsc.VectorSubcoreMesh(core_axis_name="c", subcore_axis_name="s")

    @functools.partial(
        pl.kernel,
        mesh=mesh,
        out_type=jax.ShapeDtypeStruct((_B, _L, _DIM), jnp.float32),
        scratch_types=[
            pltpu.VMEM((_RPW, _L), jnp.int32),
            pltpu.VMEM((4, _L, _DIM), jnp.float32),
            pltpu.SemaphoreType.DMA,
            pltpu.SemaphoreType.DMA,
        ],
    )
    def k(table_hbm, x_hbm, out_hbm, idx_v, rows_v, gsem, osem):
        wid = lax.axis_index("s") * _NC + lax.axis_index("c")
        base = wid * _RPW
        pltpu.sync_copy(x_hbm.at[pl.ds(base, _RPW)], idx_v)
        pltpu.async_copy(table_hbm.at[idx_v.at[0]], rows_v.at[0], gsem)
        pltpu.async_copy(table_hbm.at[idx_v.at[1]], rows_v.at[1], gsem)

        def outer(h, _):
            for b in range(4):
                g = 4 * h + b
                nxt = (b + 2) % 4

                @pl.when(g >= 2)
                def _wait_prev_write():
                    pltpu.make_async_copy(
                        rows_v.at[nxt], out_hbm.at[base], osem
                    ).wait()

                @pl.when(g + 2 < _RPW)
                def _start_next_gather():
                    pltpu.async_copy(
                        table_hbm.at[idx_v.at[g + 2]], rows_v.at[nxt], gsem
                    )

                pltpu.make_async_copy(
                    table_hbm.at[idx_v.at[g]], rows_v.at[b], gsem
                ).wait()

                @plsc.parallel_loop(0, _L * _DIM // _LANES, unroll=8)
                def _scale(j):
                    r = lax.shift_right_logical(j, 3)
                    sl = pl.ds(lax.shift_left(j & 7, 4), _LANES)
                    rows_v[b, r, sl] = rows_v[b, r, sl] * _SCALE

                pltpu.async_copy(
                    rows_v.at[b], out_hbm.at[base + g], osem
                )
            return 0

        lax.fori_loop(0, _RPW // 4, outer, 0)
        pltpu.make_async_copy(rows_v.at[2], out_hbm.at[base], osem).wait()
        pltpu.make_async_copy(rows_v.at[3], out_hbm.at[base], osem).wait()

    return k


_sc_gather = _build_sc_kernel()


def kernel(x, table):
    return _sc_gather(table, x)


# speedup vs baseline: 5.3208x; 1.0319x over previous
import functools
import math

import jax
import jax.numpy as jnp
from jax import lax
from jax.experimental import pallas as pl
from jax.experimental.pallas import tpu as pltpu
from jax.experimental.pallas import tpu_sc as plsc

_VOCAB = 100000
_DIM = 128
_B = 4096
_L = 50
_NC = 2
_NS = 16
_NW = _NC * _NS
_RPW = _B // _NW
_LANES = 16
_SCALE = math.sqrt(_DIM)


def _build_sc_kernel():
    mesh = plsc.VectorSubcoreMesh(core_axis_name="c", subcore_axis_name="s")

    @functools.partial(
        pl.kernel,
        mesh=mesh,
        out_type=jax.ShapeDtypeStruct((_B, _L, _DIM), jnp.float32),
        scratch_types=[
            pltpu.VMEM((_RPW, _L), jnp.int32),
            pltpu.VMEM((8, _L, _DIM), jnp.float32),
            pltpu.SemaphoreType.DMA,
            pltpu.SemaphoreType.DMA,
        ],
    )
    def k(table_hbm, x_hbm, out_hbm, idx_v, rows_v, gsem, osem):
        wid = lax.axis_index("s") * _NC + lax.axis_index("c")
        base = wid * _RPW
        pltpu.sync_copy(x_hbm.at[pl.ds(base, _RPW)], idx_v)
        for p in range(4):
            pltpu.async_copy(table_hbm.at[idx_v.at[p]], rows_v.at[p], gsem)

        def outer(h, _):
            for b in range(8):
                g = 8 * h + b
                nxt = (b + 4) % 8

                @pl.when(g >= 4)
                def _wait_prev_write():
                    pltpu.make_async_copy(
                        rows_v.at[nxt], out_hbm.at[base], osem
                    ).wait()

                @pl.when(g + 4 < _RPW)
                def _start_next_gather():
                    pltpu.async_copy(
                        table_hbm.at[idx_v.at[g + 4]], rows_v.at[nxt], gsem
                    )

                pltpu.make_async_copy(
                    table_hbm.at[idx_v.at[g]], rows_v.at[b], gsem
                ).wait()

                @plsc.parallel_loop(0, _L * _DIM // _LANES, unroll=8)
                def _scale(j):
                    r = lax.shift_right_logical(j, 3)
                    sl = pl.ds(lax.shift_left(j & 7, 4), _LANES)
                    rows_v[b, r, sl] = rows_v[b, r, sl] * _SCALE

                pltpu.async_copy(
                    rows_v.at[b], out_hbm.at[base + g], osem
                )
            return 0

        lax.fori_loop(0, _RPW // 8, outer, 0)
        for p in range(4, 8):
            pltpu.make_async_copy(rows_v.at[p], out_hbm.at[base], osem).wait()

    return k


_sc_gather = _build_sc_kernel()


def kernel(x, table):
    return _sc_gather(table, x)


# speedup vs baseline: 5.3236x; 1.0005x over previous
import functools
import math

import jax
import jax.numpy as jnp
from jax import lax
from jax.experimental import pallas as pl
from jax.experimental.pallas import tpu as pltpu
from jax.experimental.pallas import tpu_sc as plsc

_VOCAB = 100000
_DIM = 128
_B = 4096
_L = 50
_NC = 2
_NS = 16
_NW = _NC * _NS
_RPW = _B // _NW
_LANES = 16
_SCALE = math.sqrt(_DIM)


def _build_sc_kernel():
    mesh = plsc.VectorSubcoreMesh(core_axis_name="c", subcore_axis_name="s")

    @functools.partial(
        pl.kernel,
        mesh=mesh,
        out_type=jax.ShapeDtypeStruct((_B, _L, _DIM), jnp.float32),
        compiler_params=pltpu.CompilerParams(use_tc_tiling_on_sc=True),
        scratch_types=[
            pltpu.VMEM((_RPW, _L), jnp.int32),
            pltpu.VMEM((8, _L, _DIM), jnp.float32),
            pltpu.SemaphoreType.DMA,
            pltpu.SemaphoreType.DMA,
        ],
    )
    def k(table_hbm, x_hbm, out_hbm, idx_v, rows_v, gsem, osem):
        wid = lax.axis_index("s") * _NC + lax.axis_index("c")
        base = wid * _RPW
        pltpu.sync_copy(x_hbm.at[pl.ds(base, _RPW)], idx_v)
        for p in range(4):
            pltpu.async_copy(table_hbm.at[idx_v.at[p]], rows_v.at[p], gsem)

        def outer(h, _):
            for b in range(8):
                g = 8 * h + b
                nxt = (b + 4) % 8

                @pl.when(g >= 4)
                def _wait_prev_write():
                    pltpu.make_async_copy(
                        rows_v.at[nxt], out_hbm.at[base], osem
                    ).wait()

                @pl.when(g + 4 < _RPW)
                def _start_next_gather():
                    pltpu.async_copy(
                        table_hbm.at[idx_v.at[g + 4]], rows_v.at[nxt], gsem
                    )

                pltpu.make_async_copy(
                    table_hbm.at[idx_v.at[g]], rows_v.at[b], gsem
                ).wait()

                @plsc.parallel_loop(0, _L * _DIM // _LANES, unroll=8)
                def _scale(j):
                    r = lax.shift_right_logical(j, 3)
                    sl = pl.ds(lax.shift_left(j & 7, 4), _LANES)
                    rows_v[b, r, sl] = rows_v[b, r, sl] * _SCALE

                pltpu.async_copy(
                    rows_v.at[b], out_hbm.at[base + g], osem
                )
            return 0

        lax.fori_loop(0, _RPW // 8, outer, 0)
        for p in range(4, 8):
            pltpu.make_async_copy(rows_v.at[p], out_hbm.at[base], osem).wait()

    return k


_sc_gather = _build_sc_kernel()


def kernel(x, table):
    return _sc_gather(table, x)


# speedup vs baseline: 9.5579x; 1.7954x over previous
import functools
import math

import jax
import jax.numpy as jnp
from jax import lax
from jax.experimental import pallas as pl
from jax.experimental.pallas import tpu as pltpu
from jax.experimental.pallas import tpu_sc as plsc

_VOCAB = 100000
_DIM = 128
_B = 4096
_L = 50
_NC = 2
_NS = 16
_NW = _NC * _NS
_RPW = _B // _NW
_LANES = 16
_SCALE = math.sqrt(_DIM)
_RING = 5


def _build_sc_kernel():
    mesh = plsc.VectorSubcoreMesh(core_axis_name="c", subcore_axis_name="s")

    @functools.partial(
        pl.kernel,
        mesh=mesh,
        out_type=jax.ShapeDtypeStruct((_L, _B, _DIM), jnp.float32),
        scratch_types=[
            pltpu.VMEM((_L, _RPW), jnp.int32),
            pltpu.VMEM((_RING, _RPW, _DIM), jnp.float32),
            pltpu.SemaphoreType.DMA,
            pltpu.SemaphoreType.DMA,
        ],
    )
    def k(table_hbm, xt_hbm, out_hbm, idx_v, rows_v, gsem, osem):
        wid = lax.axis_index("s") * _NC + lax.axis_index("c")
        base = wid * _RPW
        pltpu.sync_copy(xt_hbm.at[:, pl.ds(base, _RPW)], idx_v)
        for p in range(2):
            pltpu.async_copy(table_hbm.at[idx_v.at[p]], rows_v.at[p], gsem)

        def outer(h, _):
            for b in range(_RING):
                l = _RING * h + b
                nxt = (b + 2) % _RING

                @pl.when(l >= 3)
                def _wait_prev_write():
                    pltpu.make_async_copy(
                        rows_v.at[nxt], out_hbm.at[0, pl.ds(base, _RPW)], osem
                    ).wait()

                @pl.when(l + 2 < _L)
                def _start_next_gather():
                    pltpu.async_copy(
                        table_hbm.at[idx_v.at[l + 2]], rows_v.at[nxt], gsem
                    )

                pltpu.make_async_copy(
                    table_hbm.at[idx_v.at[l]], rows_v.at[b], gsem
                ).wait()

                @plsc.parallel_loop(0, _RPW * _DIM // _LANES, unroll=8)
                def _scale(j):
                    r = lax.shift_right_logical(j, 3)
                    sl = pl.ds(lax.shift_left(j & 7, 4), _LANES)
                    rows_v[b, r, sl] = rows_v[b, r, sl] * _SCALE

                pltpu.async_copy(
                    rows_v.at[b], out_hbm.at[l, pl.ds(base, _RPW)], osem
                )
            return 0

        lax.fori_loop(0, _L // _RING, outer, 0)
        for p in range(3):
            pltpu.make_async_copy(
                rows_v.at[p], out_hbm.at[0, pl.ds(base, _RPW)], osem
            ).wait()

    return k


_sc_gather = _build_sc_kernel()


def kernel(x, table):
    out_lbd = _sc_gather(table, x.T)
    return out_lbd.transpose(1, 0, 2)
